# baseline (speedup 1.0000x reference)
import functools

import jax
import jax.numpy as jnp
from jax import lax
from jax.experimental import pallas as pl
from jax.experimental.pallas import tpu as pltpu
from jax.experimental.pallas import tpu_sc as plsc


def _sc_body(num_cores, rows_per_w, table_hbm, out_hbm, buf, sem):
    wid = lax.axis_index("s") * num_cores + lax.axis_index("c")
    base = wid * rows_per_w
    pltpu.sync_copy(table_hbm.at[pl.ds(base, rows_per_w)], buf)
    pltpu.sync_copy(buf, out_hbm.at[0, pl.ds(base, rows_per_w)])


@functools.cache
def _make_sc(num_rows, d_model, dtype):
    info = plsc.get_sparse_core_info()
    num_workers = info.num_cores * info.num_subcores
    rows_per_w = num_rows // num_workers
    mesh = plsc.VectorSubcoreMesh(core_axis_name="c", subcore_axis_name="s")
    return pl.kernel(
        functools.partial(_sc_body, info.num_cores, rows_per_w),
        mesh=mesh,
        out_type=jax.ShapeDtypeStruct((1, num_rows, d_model), dtype),
        scratch_types=[
            pltpu.VMEM((rows_per_w, d_model), dtype),
            pltpu.SemaphoreType.DMA,
        ],
    )


def _tc_body(nb, in_ref, out_ref):
    t = in_ref[...]
    for b in range(nb):
        out_ref[b] = t


@functools.cache
def _make_tc(nb, num_rows, d_model, dtype, block_rows=256):
    grid = (num_rows // block_rows,)
    return pl.pallas_call(
        functools.partial(_tc_body, nb),
        grid=grid,
        in_specs=[pl.BlockSpec((block_rows, d_model), lambda i: (i, 0))],
        out_specs=pl.BlockSpec((nb, block_rows, d_model), lambda i: (0, i, 0)),
        out_shape=jax.ShapeDtypeStruct((nb, num_rows, d_model), dtype),
    )


def kernel(x, table):
    batch, seq_len = x.shape
    num_rows, d_model = table.shape
    sc_out = _make_sc(seq_len, d_model, table.dtype)(table)
    tc_out = _make_tc(batch - 1, seq_len, d_model, table.dtype)(table)
    return jnp.concatenate([sc_out, tc_out], axis=0)
